# Initial kernel scaffold; baseline (speedup 1.0000x reference)
import jax
import jax.numpy as jnp
from jax.experimental import pallas as pl


def kernel(x, W_enc, b_enc, gamma, beta, W_dec, b_dec):
    raise NotImplementedError("write your pallas kernel here")


# speedup vs baseline: 25.2275x; 25.2275x over previous
import functools

import jax
import jax.numpy as jnp
from jax.experimental import pallas as pl
from jax.experimental.pallas import tpu as pltpu

_K = 50
_BISECT_ITERS = 18


def _sae_block(x_ref, we_ref, be_ref, g_ref, bt_ref, wd_ref, bd_ref,
               recon_ref, code_ref, *, k):
    x = x_ref[...].astype(jnp.bfloat16)
    h = jnp.dot(x, we_ref[...], preferred_element_type=jnp.float32)
    h = h + be_ref[...]
    mu = jnp.mean(h, axis=-1, keepdims=True)
    var = jnp.mean((h - mu) * (h - mu), axis=-1, keepdims=True)
    hn = (h - mu) * jax.lax.rsqrt(var + 1e-5) * g_ref[...] + bt_ref[...]
    code = jnp.maximum(hn, 0.0)

    kf = jnp.float32(k)

    def count_ge(t):
        return jnp.sum((code >= t).astype(jnp.float32), axis=-1, keepdims=True)

    bm = code.shape[0]
    lo = jnp.zeros((bm, 1), jnp.float32)
    hi = jnp.full((bm, 1), 1024.0, jnp.float32)

    c1 = count_ge(jnp.float32(1.655))
    ge1 = c1 >= kf
    lo = jnp.where(ge1, 1.655, lo)
    hi = jnp.where(ge1, hi, 1.655)
    t2 = jnp.where(ge1, 2.2, 1.15)
    c2 = count_ge(t2)
    ge2 = c2 >= kf
    lo = jnp.where(ge2, t2, lo)
    hi = jnp.where(ge2, hi, t2)

    for _ in range(_BISECT_ITERS):
        mid = (lo + hi) * 0.5
        ge = count_ge(mid) >= kf
        lo = jnp.where(ge, mid, lo)
        hi = jnp.where(ge, hi, mid)

    code = jnp.where(code >= lo, code, 0.0)
    code_ref[...] = code

    r = jnp.dot(code.astype(jnp.bfloat16), wd_ref[...],
                preferred_element_type=jnp.float32)
    recon_ref[...] = jnp.tanh(r + bd_ref[...])


def kernel(x, W_enc, b_enc, gamma, beta, W_dec, b_dec):
    B, D = x.shape
    H = W_enc.shape[1]
    bm = 512
    grid = (B // bm,)

    we_bf = W_enc.astype(jnp.bfloat16)
    wd_bf = W_dec.astype(jnp.bfloat16)
    be2 = b_enc.reshape(1, H)
    g2 = gamma.reshape(1, H)
    bt2 = beta.reshape(1, H)
    bd2 = b_dec.reshape(1, D)

    recon, code = pl.pallas_call(
        functools.partial(_sae_block, k=_K),
        grid=grid,
        in_specs=[
            pl.BlockSpec((bm, D), lambda i: (i, 0)),
            pl.BlockSpec((D, H), lambda i: (0, 0)),
            pl.BlockSpec((1, H), lambda i: (0, 0)),
            pl.BlockSpec((1, H), lambda i: (0, 0)),
            pl.BlockSpec((1, H), lambda i: (0, 0)),
            pl.BlockSpec((H, D), lambda i: (0, 0)),
            pl.BlockSpec((1, D), lambda i: (0, 0)),
        ],
        out_specs=[
            pl.BlockSpec((bm, D), lambda i: (i, 0)),
            pl.BlockSpec((bm, H), lambda i: (i, 0)),
        ],
        out_shape=[
            jax.ShapeDtypeStruct((B, D), jnp.float32),
            jax.ShapeDtypeStruct((B, H), jnp.float32),
        ],
        compiler_params=pltpu.CompilerParams(
            dimension_semantics=("arbitrary",),
        ),
    )(x, we_bf, be2, g2, bt2, wd_bf, bd2)
    return (recon, code)
